# baseline (speedup 1.0000x reference)
import numpy as np
import jax
import jax.numpy as jnp
from jax import lax
from jax.experimental import pallas as pl
from jax.experimental.pallas import tpu as pltpu

_NPOS, _NPD1, _NPD2, _NCT, _NCF = 15, 21, 11, 9, 10
_SEG = ((0, 15), (15, 36), (36, 47), (47, 56), (56, 66), (66, 68))
_KANA_OFF = 66
_NROWS = 68
_RPAD = 128
_IN_DIM = 176
_HID = 512

_lane = np.arange(_RPAD)
_SEL = np.stack([((_lane >= a) & (_lane < b)) for a, b in _SEG]
                ).astype(np.float32)
_MAXOFF = np.array(
    [[_NPOS - 1, _NPD1 - 1, _NPD2 - 1, _NCT - 1, _NCF - 1, 0, 0, 0],
     [0, 15, 36, 47, 56, 0, 0, 0]], dtype=np.int32)


def _nt(a, b):
    return lax.dot_general(a, b, (((1,), (1,)), ((), ())),
                           preferred_element_type=jnp.float32)


def _prep_body(pos_ref, pd1_ref, pd2_ref, ct_ref, cf_ref, kana_ref,
               kpw_ref, kpb_ref, cpw_ref, cpb_ref, lng_ref, lnb_ref,
               pw_ref, pb_ref,
               t_ref, sq_ref, wgc_ref, cpw11_ref, cpb_o, wsum_ref, bias_ref,
               e_scr):
    Wg = pw_ref[...] * lng_ref[...].reshape(1, _IN_DIM)
    ekana1 = _nt(kana_ref[1:2, :], kpw_ref[...]) + kpb_ref[...].reshape(1, 32)
    ekana0 = kpb_ref[...].reshape(1, 32)
    e_scr[...] = jnp.zeros((_NROWS, _IN_DIM), dtype=jnp.float32)
    e_scr[0:15, 0:32] = pos_ref[...]
    e_scr[15:36, 32:64] = pd1_ref[...]
    e_scr[36:47, 64:80] = pd2_ref[...]
    e_scr[47:56, 80:96] = ct_ref[...]
    e_scr[56:66, 96:112] = cf_ref[...]
    e_scr[66:67, 112:144] = ekana0
    e_scr[67:68, 112:144] = ekana1
    E = e_scr[...]
    t_ref[...] = jnp.zeros((_RPAD, _HID), dtype=jnp.float32)
    t_ref[0:_NROWS, :] = _nt(E, Wg)
    s = jnp.sum(E, axis=1, keepdims=True)
    q = jnp.sum(E * E, axis=1, keepdims=True)
    sq_ref[...] = jnp.zeros((_RPAD, 8), dtype=jnp.float32)
    sq_ref[0:_NROWS, 0:1] = s
    sq_ref[0:_NROWS, 1:2] = q
    wgc_ref[...] = Wg[:, 144:176]
    cpw11_ref[...] = jnp.zeros((32, 11), dtype=jnp.float32)
    cpw11_ref[:, 5:11] = cpw_ref[...]
    cpb_o[...] = cpb_ref[...].reshape(1, 32)
    ones = jnp.ones((1, _IN_DIM), dtype=jnp.float32)
    wsum_ref[...] = _nt(ones, Wg)
    bias_ref[...] = (_nt(lnb_ref[...].reshape(1, _IN_DIM), pw_ref[...])
                     + pb_ref[...].reshape(1, _HID))


def _body(x_ref, rid_ref, mo_ref, sel_ref, t_ref, sq_ref, wgc_ref,
          cpw11_ref, cpb_ref, wsum_ref, bias_ref, out_ref):
    bb = x_ref.shape[0]
    xb = x_ref[...].reshape(bb * 50, 11)
    rid = rid_ref[...].reshape(bb * 50, 8)
    bm = xb.shape[0]
    cat = xb[:, 0:5].astype(jnp.int32)

    maxv = mo_ref[0:1, 0:5]
    offs = mo_ref[1:2, 0:5]
    flat = jnp.clip(cat, 0, maxv) + offs

    kmax = jnp.max(rid, axis=1, keepdims=True)
    kana_row = _KANA_OFF + (kmax >= 1).astype(jnp.int32)
    flat6 = jnp.concatenate([flat, kana_row], axis=1).astype(jnp.float32)

    spread = jnp.dot(flat6, sel_ref[...], preferred_element_type=jnp.float32)
    iota = lax.broadcasted_iota(jnp.int32, (bm, _RPAD), 1).astype(jnp.float32)
    f = (spread == iota).astype(jnp.float32)

    e_cont = _nt(xb, cpw11_ref[...]) + cpb_ref[...]
    s_cont = jnp.sum(e_cont, axis=1, keepdims=True)
    q_cont = jnp.sum(e_cont * e_cont, axis=1, keepdims=True)

    fs = jnp.dot(f, sq_ref[...], preferred_element_type=jnp.float32)
    s1 = fs[:, 0:1] + s_cont
    s2 = fs[:, 1:2] + q_cont
    mu = s1 * (1.0 / _IN_DIM)
    var = s2 * (1.0 / _IN_DIM) - mu * mu
    rstd = lax.rsqrt(var + 1e-5)

    acc = jnp.dot(f, t_ref[...], preferred_element_type=jnp.float32)
    acc = acc + _nt(e_cont, wgc_ref[...])
    out_ref[...] = (acc - mu * wsum_ref[...]) * rstd + bias_ref[...]


def kernel(x, reading_ids, pos_emb, pd1_emb, pd2_emb, ct_emb, cf_emb,
           kana_emb, kana_proj_w, kana_proj_b, cont_proj_w, cont_proj_b,
           ln_g, ln_b, proj_w, proj_b):
    B, S, _ = x.shape

    full = lambda *shape: pl.BlockSpec(shape, lambda: tuple(0 for _ in shape))
    T, sq, wgc, cpw11, cpb_r, wsumg, bias = pl.pallas_call(
        _prep_body,
        in_specs=[full(15, 32), full(21, 32), full(11, 16), full(9, 16),
                  full(10, 16), full(2, 32), full(32, 32), full(32,),
                  full(32, 6), full(32,), full(176,), full(176,),
                  full(512, 176), full(512,)],
        out_specs=[full(_RPAD, _HID), full(_RPAD, 8), full(_HID, 32),
                   full(32, 11), full(1, 32), full(1, _HID), full(1, _HID)],
        out_shape=[
            jax.ShapeDtypeStruct((_RPAD, _HID), jnp.float32),
            jax.ShapeDtypeStruct((_RPAD, 8), jnp.float32),
            jax.ShapeDtypeStruct((_HID, 32), jnp.float32),
            jax.ShapeDtypeStruct((32, 11), jnp.float32),
            jax.ShapeDtypeStruct((1, 32), jnp.float32),
            jax.ShapeDtypeStruct((1, _HID), jnp.float32),
            jax.ShapeDtypeStruct((1, _HID), jnp.float32),
        ],
        scratch_shapes=[pltpu.VMEM((_NROWS, _IN_DIM), jnp.float32)],
    )(pos_emb, pd1_emb, pd2_emb, ct_emb, cf_emb, kana_emb, kana_proj_w,
      kana_proj_b, cont_proj_w, cont_proj_b, ln_g, ln_b, proj_w, proj_b)

    bb = 16
    grid = (B // bb,)
    out = pl.pallas_call(
        _body,
        grid=grid,
        in_specs=[
            pl.BlockSpec((bb, S, 11), lambda i: (i, 0, 0)),
            pl.BlockSpec((bb, S, 8), lambda i: (i, 0, 0)),
            pl.BlockSpec((2, 8), lambda i: (0, 0)),
            pl.BlockSpec((6, _RPAD), lambda i: (0, 0)),
            pl.BlockSpec((_RPAD, _HID), lambda i: (0, 0)),
            pl.BlockSpec((_RPAD, 8), lambda i: (0, 0)),
            pl.BlockSpec((_HID, 32), lambda i: (0, 0)),
            pl.BlockSpec((32, 11), lambda i: (0, 0)),
            pl.BlockSpec((1, 32), lambda i: (0, 0)),
            pl.BlockSpec((1, _HID), lambda i: (0, 0)),
            pl.BlockSpec((1, _HID), lambda i: (0, 0)),
        ],
        out_specs=pl.BlockSpec((bb * S, _HID), lambda i: (i, 0)),
        out_shape=jax.ShapeDtypeStruct((B * S, _HID), jnp.float32),
    )(x, reading_ids, jnp.asarray(_MAXOFF), jnp.asarray(_SEL), T, sq,
      wgc, cpw11, cpb_r, wsumg, bias)
    return out.reshape(B, S, _HID)


# speedup vs baseline: 1.3725x; 1.3725x over previous
import numpy as np
import jax
import jax.numpy as jnp
from jax import lax
from jax.experimental import pallas as pl
from jax.experimental.pallas import tpu as pltpu

_NPOS, _NPD1, _NPD2, _NCT, _NCF = 15, 21, 11, 9, 10
_SEG = ((0, 15), (15, 36), (36, 47), (47, 56), (56, 66), (66, 68))
_KANA_OFF = 66
_NROWS = 68
_RPAD = 128
_IN_DIM = 176
_HID = 512

_lane = np.arange(_RPAD)
_SEL = np.stack([((_lane >= a) & (_lane < b)) for a, b in _SEG]
                ).astype(np.float32)
_MAXOFF = np.array(
    [[_NPOS - 1, _NPD1 - 1, _NPD2 - 1, _NCT - 1, _NCF - 1, 0, 0, 0],
     [0, 15, 36, 47, 56, 0, 0, 0]], dtype=np.int32)


def _nt(a, b):
    return lax.dot_general(a, b, (((1,), (1,)), ((), ())),
                           preferred_element_type=jnp.float32)


def _prep_body(pos_ref, pd1_ref, pd2_ref, ct_ref, cf_ref, kana_ref,
               kpw_ref, kpb_ref, cpw_ref, cpb_ref, lng_ref, lnb_ref,
               pw_ref, pb_ref,
               t_ref, sq_ref, wgc_ref, cpw11_ref, cpb_o, wsum_ref, bias_ref,
               e_scr):
    Wg = pw_ref[...] * lng_ref[...].reshape(1, _IN_DIM)
    ekana1 = _nt(kana_ref[1:2, :], kpw_ref[...]) + kpb_ref[...].reshape(1, 32)
    ekana0 = kpb_ref[...].reshape(1, 32)
    e_scr[...] = jnp.zeros((_NROWS, _IN_DIM), dtype=jnp.float32)
    e_scr[0:15, 0:32] = pos_ref[...]
    e_scr[15:36, 32:64] = pd1_ref[...]
    e_scr[36:47, 64:80] = pd2_ref[...]
    e_scr[47:56, 80:96] = ct_ref[...]
    e_scr[56:66, 96:112] = cf_ref[...]
    e_scr[66:67, 112:144] = ekana0
    e_scr[67:68, 112:144] = ekana1
    E = e_scr[...]
    t_ref[...] = jnp.zeros((_RPAD, _HID), dtype=jnp.float32)
    t_ref[0:_NROWS, :] = _nt(E, Wg)
    s = jnp.sum(E, axis=1, keepdims=True)
    q = jnp.sum(E * E, axis=1, keepdims=True)
    sq_ref[...] = jnp.zeros((_RPAD, 8), dtype=jnp.float32)
    sq_ref[0:_NROWS, 0:1] = s
    sq_ref[0:_NROWS, 1:2] = q
    wgc_ref[...] = Wg[:, 144:176]
    cpw11_ref[...] = jnp.zeros((32, 11), dtype=jnp.float32)
    cpw11_ref[:, 5:11] = cpw_ref[...]
    cpb_o[...] = cpb_ref[...].reshape(1, 32)
    ones = jnp.ones((1, _IN_DIM), dtype=jnp.float32)
    wsum_ref[...] = _nt(ones, Wg)
    bias_ref[...] = (_nt(lnb_ref[...].reshape(1, _IN_DIM), pw_ref[...])
                     + pb_ref[...].reshape(1, _HID))


def _body(x_ref, rid_ref, mo_ref, sel_ref, t_ref, sq_ref, wgc_ref,
          cpw11_ref, cpb_ref, wsum_ref, bias_ref, out_ref):
    bb = x_ref.shape[0]
    xb = x_ref[...].reshape(bb * 50, 11)
    rid = rid_ref[...].reshape(bb * 50, 8)
    bm = xb.shape[0]
    cat = xb[:, 0:5].astype(jnp.int32)

    maxv = mo_ref[0:1, 0:5]
    offs = mo_ref[1:2, 0:5]
    flat = jnp.clip(cat, 0, maxv) + offs

    kmax = jnp.max(rid, axis=1, keepdims=True)
    kana_row = _KANA_OFF + (kmax >= 1).astype(jnp.int32)
    flat6 = jnp.concatenate([flat, kana_row], axis=1).astype(jnp.float32)

    spread = jnp.dot(flat6, sel_ref[...], preferred_element_type=jnp.float32)
    iota = lax.broadcasted_iota(jnp.int32, (bm, _RPAD), 1).astype(jnp.float32)
    f = (spread == iota).astype(jnp.float32)

    e_cont = _nt(xb, cpw11_ref[...]) + cpb_ref[...]
    s_cont = jnp.sum(e_cont, axis=1, keepdims=True)
    q_cont = jnp.sum(e_cont * e_cont, axis=1, keepdims=True)

    fs = jnp.dot(f, sq_ref[...], preferred_element_type=jnp.float32)
    s1 = fs[:, 0:1] + s_cont
    s2 = fs[:, 1:2] + q_cont
    mu = s1 * (1.0 / _IN_DIM)
    var = s2 * (1.0 / _IN_DIM) - mu * mu
    rstd = lax.rsqrt(var + 1e-5)

    acc = jnp.dot(f, t_ref[...], preferred_element_type=jnp.float32)
    acc = acc + _nt(e_cont, wgc_ref[...])
    res = (acc - mu * wsum_ref[...]) * rstd + bias_ref[...]
    out_ref[...] = res.reshape(bb, 50, _HID)


def kernel(x, reading_ids, pos_emb, pd1_emb, pd2_emb, ct_emb, cf_emb,
           kana_emb, kana_proj_w, kana_proj_b, cont_proj_w, cont_proj_b,
           ln_g, ln_b, proj_w, proj_b):
    B, S, _ = x.shape

    full = lambda *shape: pl.BlockSpec(shape, lambda: tuple(0 for _ in shape))
    T, sq, wgc, cpw11, cpb_r, wsumg, bias = pl.pallas_call(
        _prep_body,
        in_specs=[full(15, 32), full(21, 32), full(11, 16), full(9, 16),
                  full(10, 16), full(2, 32), full(32, 32), full(32,),
                  full(32, 6), full(32,), full(176,), full(176,),
                  full(512, 176), full(512,)],
        out_specs=[full(_RPAD, _HID), full(_RPAD, 8), full(_HID, 32),
                   full(32, 11), full(1, 32), full(1, _HID), full(1, _HID)],
        out_shape=[
            jax.ShapeDtypeStruct((_RPAD, _HID), jnp.float32),
            jax.ShapeDtypeStruct((_RPAD, 8), jnp.float32),
            jax.ShapeDtypeStruct((_HID, 32), jnp.float32),
            jax.ShapeDtypeStruct((32, 11), jnp.float32),
            jax.ShapeDtypeStruct((1, 32), jnp.float32),
            jax.ShapeDtypeStruct((1, _HID), jnp.float32),
            jax.ShapeDtypeStruct((1, _HID), jnp.float32),
        ],
        scratch_shapes=[pltpu.VMEM((_NROWS, _IN_DIM), jnp.float32)],
    )(pos_emb, pd1_emb, pd2_emb, ct_emb, cf_emb, kana_emb, kana_proj_w,
      kana_proj_b, cont_proj_w, cont_proj_b, ln_g, ln_b, proj_w, proj_b)

    bb = 32
    grid = (B // bb,)
    out = pl.pallas_call(
        _body,
        grid=grid,
        in_specs=[
            pl.BlockSpec((bb, S, 11), lambda i: (i, 0, 0)),
            pl.BlockSpec((bb, S, 8), lambda i: (i, 0, 0)),
            pl.BlockSpec((2, 8), lambda i: (0, 0)),
            pl.BlockSpec((6, _RPAD), lambda i: (0, 0)),
            pl.BlockSpec((_RPAD, _HID), lambda i: (0, 0)),
            pl.BlockSpec((_RPAD, 8), lambda i: (0, 0)),
            pl.BlockSpec((_HID, 32), lambda i: (0, 0)),
            pl.BlockSpec((32, 11), lambda i: (0, 0)),
            pl.BlockSpec((1, 32), lambda i: (0, 0)),
            pl.BlockSpec((1, _HID), lambda i: (0, 0)),
            pl.BlockSpec((1, _HID), lambda i: (0, 0)),
        ],
        out_specs=pl.BlockSpec((bb, S, _HID), lambda i: (i, 0, 0)),
        out_shape=jax.ShapeDtypeStruct((B, S, _HID), jnp.float32),
    )(x, reading_ids, jnp.asarray(_MAXOFF), jnp.asarray(_SEL), T, sq,
      wgc, cpw11, cpb_r, wsumg, bias)
    return out
